# baseline (speedup 1.0000x reference)
import jax
import jax.numpy as jnp
from jax import lax
from jax.experimental import pallas as pl
from jax.experimental.pallas import tpu as pltpu
from jax.experimental.pallas import tpu_sc as plsc

N = 100000
K = 32
D = 128
B = 4096

NC = 2
NS = 16
NW = NC * NS
BW = B // NW
L = 16
DJ = D // L


def _gat_body(x_hbm, adj_hbm, tab_hbm, out_hbm,
              xv, adjv, qv, nbr0, nbr1, nbr2, nbr3, outv,
              sem_a, sem_q, sem_n0, sem_n1, sem_n2, sem_n3):
    c = lax.axis_index("c")
    s_ = lax.axis_index("s")
    wid = c * NS + s_
    base = wid * BW

    lane = lax.broadcasted_iota(jnp.int32, (L,), 0)

    pltpu.sync_copy(x_hbm.at[pl.ds(base, BW)], xv)
    cp_q = pltpu.async_copy(tab_hbm.at[xv], qv, sem_q)
    for cc in range(BW // L):
        xch = xv[pl.ds(cc * L, L)]
        for i in range(L):
            pltpu.async_copy(adj_hbm.at[xch[i]], adjv.at[cc * L + i], sem_a)
    for cc in range(BW // L):
        for i in range(L):
            pltpu.make_async_copy(
                adj_hbm.at[0], adjv.at[cc * L + i], sem_a).wait()

    nbrs = (nbr0, nbr1, nbr2, nbr3)
    sems = (sem_n0, sem_n1, sem_n2, sem_n3)
    for bb in range(4):
        pltpu.async_copy(tab_hbm.at[adjv.at[bb]], nbrs[bb], sems[bb])
    cp_q.wait()

    def step(b, nbr, sem):
        pltpu.make_async_copy(tab_hbm.at[adjv.at[b]], nbr, sem).wait()

        q = [qv[b, pl.ds(j * L, L)] for j in range(DJ)]

        l0 = jnp.zeros((L,), jnp.float32)
        l1 = jnp.zeros((L,), jnp.float32)
        for k in range(K):
            acc = q[0] * nbr[k, pl.ds(0, L)]
            for j in range(1, DJ):
                acc = acc + q[j] * nbr[k, pl.ds(j * L, L)]
            logit = jnp.sum(acc)
            if k < L:
                l0 = jnp.where(lane == k, logit, l0)
            else:
                l1 = jnp.where(lane == (k - L), logit, l1)

        m = jnp.max(jnp.maximum(l0, l1))
        e0 = jnp.exp(l0 - m)
        e1 = jnp.exp(l1 - m)
        ssum = jnp.broadcast_to(jnp.sum(e0 + e1), (L,))
        r = jnp.full((L,), 1.0, jnp.float32) / ssum
        w0 = e0 * r
        w1 = e1 * r

        outs = q
        for k in range(K):
            wk = w0[k] if k < L else w1[k - L]
            outs = [outs[j] + wk * nbr[k, pl.ds(j * L, L)] for j in range(DJ)]
        for j in range(DJ):
            outv[b, pl.ds(j * L, L)] = outs[j]

        @pl.when(b + 4 < BW)
        def _refill():
            pltpu.make_async_copy(tab_hbm.at[adjv.at[b + 4]], nbr, sem).start()

    def loop(i, _):
        b = 4 * i
        for u in range(4):
            step(b + u, nbrs[u], sems[u])
        return 0

    lax.fori_loop(0, BW // 4, loop, 0)

    pltpu.sync_copy(outv, out_hbm.at[pl.ds(base, BW)])


@jax.jit
def _gat(x, adj, table):
    mesh = plsc.VectorSubcoreMesh(core_axis_name="c", subcore_axis_name="s")
    run = pl.kernel(
        _gat_body,
        mesh=mesh,
        out_type=jax.ShapeDtypeStruct((B, D), jnp.float32),
        compiler_params=pltpu.CompilerParams(
            needs_layout_passes=False, use_tc_tiling_on_sc=True),
        scratch_types=[
            pltpu.VMEM((BW,), jnp.int32),
            pltpu.VMEM((BW, K), jnp.int32),
            pltpu.VMEM((BW, D), jnp.float32),
            pltpu.VMEM((K, D), jnp.float32),
            pltpu.VMEM((K, D), jnp.float32),
            pltpu.VMEM((K, D), jnp.float32),
            pltpu.VMEM((K, D), jnp.float32),
            pltpu.VMEM((BW, D), jnp.float32),
            pltpu.SemaphoreType.DMA,
            pltpu.SemaphoreType.DMA,
            pltpu.SemaphoreType.DMA,
            pltpu.SemaphoreType.DMA,
            pltpu.SemaphoreType.DMA,
            pltpu.SemaphoreType.DMA,
        ],
    )
    return run(x, adj, table)


def kernel(X, adj, table):
    x = X.reshape(B).astype(jnp.int32)
    out = _gat(x, adj, table)
    return out[:, None, :]


# speedup vs baseline: 1.3546x; 1.3546x over previous
import jax
import jax.numpy as jnp
from jax import lax
from jax.experimental import pallas as pl
from jax.experimental.pallas import tpu as pltpu
from jax.experimental.pallas import tpu_sc as plsc

N = 100000
K = 32
D = 128
B = 4096

NC = 2
NS = 16
NW = NC * NS
BW = B // NW
L = 16
DJ = D // L


def _gat_body(x_hbm, adj_hbm, tab_hbm, out_hbm,
              xv, adjv, qv, nbr0, nbr1, outv,
              sem_a, sem_q, sem_n0, sem_n1):
    c = lax.axis_index("c")
    s_ = lax.axis_index("s")
    wid = c * NS + s_
    base = wid * BW

    lane = lax.broadcasted_iota(jnp.int32, (L,), 0)

    pltpu.sync_copy(x_hbm.at[pl.ds(base, BW)], xv)
    cp_q = pltpu.async_copy(tab_hbm.at[xv], qv, sem_q)
    for cc in range(BW // L):
        xch = xv[pl.ds(cc * L, L)]
        for i in range(L):
            pltpu.async_copy(adj_hbm.at[xch[i]], adjv.at[cc * L + i], sem_a)
    for cc in range(BW // L):
        for i in range(L):
            pltpu.make_async_copy(
                adj_hbm.at[0], adjv.at[cc * L + i], sem_a).wait()

    pltpu.async_copy(tab_hbm.at[adjv.at[0]], nbr0, sem_n0)
    pltpu.async_copy(tab_hbm.at[adjv.at[1]], nbr1, sem_n1)
    cp_q.wait()

    nbrs = (nbr0, nbr1)
    sems = (sem_n0, sem_n1)

    def step(b, nbr, sem):
        pltpu.make_async_copy(tab_hbm.at[adjv.at[b]], nbr, sem).wait()

        q = [qv[b, pl.ds(j * L, L)] for j in range(DJ)]

        l0 = jnp.zeros((L,), jnp.float32)
        l1 = jnp.zeros((L,), jnp.float32)
        for k in range(K):
            acc = q[0] * nbr[k, pl.ds(0, L)]
            for j in range(1, DJ):
                acc = acc + q[j] * nbr[k, pl.ds(j * L, L)]
            logit = jnp.sum(acc)
            if k < L:
                l0 = jnp.where(lane == k, logit, l0)
            else:
                l1 = jnp.where(lane == (k - L), logit, l1)

        m = jnp.max(jnp.maximum(l0, l1))
        e0 = jnp.exp(l0 - m)
        e1 = jnp.exp(l1 - m)
        ssum = jnp.broadcast_to(jnp.sum(e0 + e1), (L,))
        r = jnp.full((L,), 1.0, jnp.float32) / ssum
        w0 = e0 * r
        w1 = e1 * r

        outs = q
        for k in range(K):
            wk = w0[k] if k < L else w1[k - L]
            outs = [outs[j] + wk * nbr[k, pl.ds(j * L, L)] for j in range(DJ)]
        for j in range(DJ):
            outv[b, pl.ds(j * L, L)] = outs[j]

        @pl.when(b + 2 < BW)
        def _refill():
            pltpu.make_async_copy(tab_hbm.at[adjv.at[b + 2]], nbr, sem).start()

    def loop(i, _):
        b = 2 * i
        step(b, nbrs[0], sems[0])
        step(b + 1, nbrs[1], sems[1])
        return 0

    lax.fori_loop(0, BW // 2, loop, 0)

    pltpu.sync_copy(outv, out_hbm.at[pl.ds(base, BW)])


@jax.jit
def _gat(x, adj, table):
    mesh = plsc.VectorSubcoreMesh(core_axis_name="c", subcore_axis_name="s")
    run = pl.kernel(
        _gat_body,
        mesh=mesh,
        out_type=jax.ShapeDtypeStruct((B, D), jnp.float32),
        compiler_params=pltpu.CompilerParams(
            needs_layout_passes=False, use_tc_tiling_on_sc=True),
        scratch_types=[
            pltpu.VMEM((BW,), jnp.int32),
            pltpu.VMEM((BW, K), jnp.int32),
            pltpu.VMEM((BW, D), jnp.float32),
            pltpu.VMEM((K, D), jnp.float32),
            pltpu.VMEM((K, D), jnp.float32),
            pltpu.VMEM((BW, D), jnp.float32),
            pltpu.SemaphoreType.DMA,
            pltpu.SemaphoreType.DMA,
            pltpu.SemaphoreType.DMA,
            pltpu.SemaphoreType.DMA,
        ],
    )
    return run(x, adj, table)


def kernel(X, adj, table):
    x = X.reshape(B).astype(jnp.int32)
    out = _gat(x, adj, table)
    return out[:, None, :]
